# baseline (speedup 1.0000x reference)
import functools

import jax
import jax.numpy as jnp
from jax import lax
from jax.experimental import pallas as pl
from jax.experimental.pallas import tpu as pltpu
from jax.experimental.pallas import tpu_sc as plsc

D_MODEL = 1024
NUM_EMB = 4

_NC = 2
_NS = 16
_NW = _NC * _NS

_TOTAL = 4 * 8192
_BPW = _TOTAL // _NW
_R = 32
_NCH = _BPW // _R


def _sc_body(ids_hbm, table_hbm, out_hbm, idx_v, bufs, gsemA, gsemB, ssemA, ssemB):
    wid = lax.axis_index("s") * _NC + lax.axis_index("c")
    base = wid * _BPW
    pltpu.sync_copy(ids_hbm.at[pl.ds(base, _BPW)], idx_v)

    gsems = (gsemA, gsemB)
    ssems = (ssemA, ssemB)
    gd = [None] * _NCH
    sd = [None] * _NCH

    def start_gather(i):
        b = i % 2
        gd[i] = pltpu.async_copy(
            table_hbm.at[idx_v.at[pl.ds(i * _R, _R)]], bufs.at[b], gsems[b])

    def start_scatter(i):
        b = i % 2
        sd[i] = pltpu.async_copy(
            bufs.at[b], out_hbm.at[pl.ds(base + i * _R, _R)], ssems[b])

    start_gather(0)
    for i in range(_NCH):
        if i >= 1:
            sd[i - 1].wait()
        if i + 1 < _NCH:
            start_gather(i + 1)
        gd[i].wait()
        start_scatter(i)
    sd[_NCH - 1].wait()


@jax.jit
def _sc_gather(ids_flat, table):
    mesh = plsc.VectorSubcoreMesh(
        core_axis_name="c", subcore_axis_name="s",
        num_cores=_NC, num_subcores=_NS)
    f = functools.partial(
        pl.kernel,
        out_type=jax.ShapeDtypeStruct((_TOTAL, D_MODEL), jnp.float32),
        mesh=mesh,
        scratch_types=[
            pltpu.VMEM((_BPW,), jnp.int32),
            pltpu.VMEM((2, _R, D_MODEL), jnp.float32),
            pltpu.SemaphoreType.DMA,
            pltpu.SemaphoreType.DMA,
            pltpu.SemaphoreType.DMA,
            pltpu.SemaphoreType.DMA,
        ],
    )(_sc_body)
    return f(ids_flat, table)


def kernel(postion_ids, table):
    B, S = postion_ids.shape
    ids_flat = postion_ids.reshape(B * S).astype(jnp.int32)
    out = _sc_gather(ids_flat, table)
    return out.reshape(B, S, D_MODEL)


# speedup vs baseline: 3.0889x; 3.0889x over previous
import functools

import jax
import jax.numpy as jnp
from jax import lax
from jax.experimental import pallas as pl
from jax.experimental.pallas import tpu as pltpu
from jax.experimental.pallas import tpu_sc as plsc

D_MODEL = 1024
NUM_EMB = 4

_NC = 2
_NS = 16
_NW = _NC * _NS

_TOTAL = 4 * 8192
_BPW = _TOTAL // _NW
_R = 32
_NCH = _BPW // _R


def _sc_body(ids_hbm, table_hbm, out_hbm, idx_v, bufs,
             gsemA, gsemB, ssemA, ssemB):
    sid = lax.axis_index("s")
    wid = sid * _NC + lax.axis_index("c")
    base = wid * _BPW

    pltpu.sync_copy(ids_hbm.at[pl.ds(base, _BPW)], idx_v)
    off = wid * NUM_EMB
    for j in range(_BPW // 16):
        sl = pl.ds(j * 16, 16)
        idx_v[sl] = idx_v[sl] + off

    gsems = (gsemA, gsemB)
    ssems = (ssemA, ssemB)
    gd = [None] * _NCH
    sd = [None] * _NCH

    def start_gather(i):
        b = i % 2
        gd[i] = pltpu.async_copy(
            table_hbm.at[idx_v.at[pl.ds(i * _R, _R)]], bufs.at[b], gsems[b])

    def start_scatter(i):
        b = i % 2
        sd[i] = pltpu.async_copy(
            bufs.at[b], out_hbm.at[pl.ds(base + i * _R, _R)], ssems[b])

    start_gather(0)
    for i in range(_NCH):
        if i >= 1:
            sd[i - 1].wait()
        if i + 1 < _NCH:
            start_gather(i + 1)
        gd[i].wait()
        start_scatter(i)
    sd[_NCH - 1].wait()


@jax.jit
def _sc_gather(ids_flat, table):
    mesh = plsc.VectorSubcoreMesh(
        core_axis_name="c", subcore_axis_name="s",
        num_cores=_NC, num_subcores=_NS)
    f = functools.partial(
        pl.kernel,
        out_type=jax.ShapeDtypeStruct((_TOTAL, D_MODEL), jnp.float32),
        mesh=mesh,
        scratch_types=[
            pltpu.VMEM((_BPW,), jnp.int32),
            pltpu.VMEM((2, _R, D_MODEL), jnp.float32),
            pltpu.SemaphoreType.DMA,
            pltpu.SemaphoreType.DMA,
            pltpu.SemaphoreType.DMA,
            pltpu.SemaphoreType.DMA,
        ],
    )(_sc_body)
    return f(ids_flat, table)


def kernel(postion_ids, table):
    B, S = postion_ids.shape
    ids_flat = postion_ids.reshape(B * S).astype(jnp.int32)
    table_rep = jnp.tile(table, (_NW, 1))
    out = _sc_gather(ids_flat, table_rep)
    return out.reshape(B, S, D_MODEL)
